# baseline (speedup 1.0000x reference)
import functools

import jax
import jax.numpy as jnp
from jax import lax
from jax.experimental import pallas as pl
from jax.experimental.pallas import tpu as pltpu
from jax.experimental.pallas import tpu_sc as plsc

_K = 128
_N = 32768
_L = 16
_HI0 = 0x3F800001
_NB = 128
_SEG2 = _N // (_L * 4)
_U = 4


def _process_row(row, row_v, cand_v, tie_v, hist_v, k_s, scores_hbm, out_hbm):
    pltpu.sync_copy(scores_hbm.at[row], row_v)
    iota = lax.iota(jnp.int32, _L)
    ones_i = jnp.ones((_L,), jnp.int32)
    zero_i = jnp.zeros((_L,), jnp.int32)

    @plsc.parallel_loop(0, _NB, unroll=4)
    def _(i):
        hist_v[pl.ds(i * _L, _L)] = zero_i

    @plsc.parallel_loop(
        0, _N // (_L * _U), unroll=2,
        carry=tuple(jnp.zeros((_L,), jnp.float32) for _ in range(_U)))
    def saccs(i, saccs):
        out = []
        for u in range(_U):
            x = row_v[pl.ds((i * _U + u) * _L, _L)]
            xc = jnp.clip(x, 0.0, 1.0)
            out.append(saccs[u] + xc)
            e = plsc.bitcast(1.0 - xc, jnp.int32) >> 23
            plsc.addupdate_scatter(hist_v, [(e << 4) + iota], ones_i)
        return tuple(out)

    s_row = jnp.sum(sum(saccs))

    carry = jnp.int32(0)
    bstar = jnp.int32(0)
    base = jnp.int32(0)
    for g in range(_NB // _L):
        tot = zero_i
        bidx = (g * _L + iota) << 4
        for p in range(_L):
            tot = tot + plsc.load_gather(hist_v, [bidx + p])
        cum = plsc.cumsum(tot) + carry
        carry = cum[_L - 1]
        below = cum < _K
        bstar = bstar + plsc.all_reduce_population_count(below)[0]
        base = jnp.maximum(base, jnp.max(jnp.where(below, cum, 0)))

    seg_bases = [(iota * _U + u) * _SEG2 for u in range(_U)]

    @plsc.parallel_loop(
        0, _N // (_L * _U), unroll=2,
        carry=tuple(zero_i for _ in range(_U)))
    def percnts(i, percnts):
        out = []
        for u in range(_U):
            x = row_v[pl.ds((i * _U + u) * _L, _L)]
            xc = jnp.clip(x, 0.0, 1.0)
            e = plsc.bitcast(1.0 - xc, jnp.int32) >> 23
            m = e == bstar
            plsc.store_scatter(cand_v, [seg_bases[u] + percnts[u]], xc, mask=m)
            out.append(percnts[u] + jnp.where(m, 1, 0))
        return tuple(out)

    cs_max = jnp.max(jnp.maximum(jnp.maximum(percnts[0], percnts[1]),
                                 jnp.maximum(percnts[2], percnts[3])))
    cs_tot = jnp.sum(percnts[0] + percnts[1] + percnts[2] + percnts[3])

    def bit(carryv):
        lo, hi, cnt_lo, cnt_hi = carryv
        mid = (lo + hi) >> 1

        def cbody(i, cacc):
            for u in range(_U):
                xb = plsc.bitcast(
                    plsc.load_gather(cand_v, [seg_bases[u] + i]), jnp.int32)
                valid = i < percnts[u]
                cacc = cacc + jnp.where(valid & (xb >= mid), 1, 0)
            return cacc

        cnt = jnp.sum(lax.fori_loop(0, cs_max, cbody, zero_i)) + base
        ge = cnt >= _K
        return (jnp.where(ge, mid, lo), jnp.where(ge, hi, mid),
                jnp.where(ge, cnt, cnt_lo), jnp.where(ge, cnt_hi, cnt))

    lo, hi, cnt_lo, cnt_hi = lax.while_loop(
        lambda c: c[1] - c[0] > 1, bit,
        (jnp.int32(0), jnp.int32(_HI0), base + cs_tot, jnp.int32(0)))

    t = lo
    n_gt = cnt_hi
    need = _K - n_gt
    cond = s_row > k_s

    @pl.when(cond & (cnt_lo == _K))
    def _():
        @plsc.parallel_loop(0, _N // (_L * _U), unroll=2)
        def _(i):
            for u in range(_U):
                sl = pl.ds((i * _U + u) * _L, _L)
                xc = jnp.clip(row_v[sl], 0.0, 1.0)
                xb = plsc.bitcast(xc, jnp.int32)
                cand_v[sl] = jnp.where(xb >= t, xc, 0.0)

    @pl.when(cond & (cnt_lo != _K))
    def _():
        def obody(i, wv):
            x = row_v[pl.ds(i * _L, _L)]
            xc = jnp.clip(x, 0.0, 1.0)
            xb = plsc.bitcast(xc, jnp.int32)
            meq = xb == t
            csum = plsc.cumsum(jnp.where(meq, 1, 0))
            tix = wv + csum - 1
            plsc.store_scatter(tie_v, [tix], i * _L + iota,
                               mask=meq & (tix < _K))
            cand_v[pl.ds(i * _L, _L)] = jnp.where(xb > t, xc, 0.0)
            return wv + plsc.all_reduce_population_count(meq)

        lax.fori_loop(0, _N // _L, obody, zero_i)
        tvals = plsc.bitcast(jnp.broadcast_to(t, (_L,)), jnp.float32)

        def rbody(i, c):
            tix = tie_v[pl.ds(i * _L, _L)]
            valid = (i * _L + iota) < need
            plsc.store_scatter(cand_v, [tix], tvals, mask=valid)
            return c

        lax.fori_loop(0, (need + _L - 1) // _L, rbody, 0)

    @pl.when(jnp.logical_not(cond))
    def _():
        @plsc.parallel_loop(0, _N // (_L * _U), unroll=2)
        def _(i):
            for u in range(_U):
                sl = pl.ds((i * _U + u) * _L, _L)
                cand_v[sl] = jnp.clip(row_v[sl], 0.0, 1.0)

    pltpu.sync_copy(cand_v, out_hbm.at[row])


def _sc_body(scores_hbm, kvec_hbm, out_hbm, row_v, cand_v, tie_v, hist_v, kv_v):
    wid = lax.axis_index("s") * 2 + lax.axis_index("c")
    pltpu.sync_copy(kvec_hbm, kv_v)
    k_s = jnp.sum(kv_v[...]) * 0.0625
    for r in range(2):
        _process_row(wid * 2 + r, row_v, cand_v, tie_v, hist_v, k_s,
                     scores_hbm, out_hbm)


def kernel(scores, k):
    kvec = jnp.broadcast_to(jnp.asarray(k, jnp.float32), (_L,))
    mesh = plsc.VectorSubcoreMesh(core_axis_name="c", subcore_axis_name="s")
    fn = functools.partial(
        pl.kernel,
        mesh=mesh,
        out_type=jax.ShapeDtypeStruct(scores.shape, scores.dtype),
        scratch_types=[
            pltpu.VMEM((_N,), jnp.float32),
            pltpu.VMEM((_N,), jnp.float32),
            pltpu.VMEM((_K + _L,), jnp.int32),
            pltpu.VMEM((_NB * _L,), jnp.int32),
            pltpu.VMEM((_L,), jnp.float32),
        ],
        compiler_params=pltpu.CompilerParams(needs_layout_passes=False),
    )(_sc_body)
    return fn(scores, kvec)


# speedup vs baseline: 1.1302x; 1.1302x over previous
import functools

import jax
import jax.numpy as jnp
from jax import lax
from jax.experimental import pallas as pl
from jax.experimental.pallas import tpu as pltpu
from jax.experimental.pallas import tpu_sc as plsc

_K = 128
_N = 32768
_L = 16
_HI0 = 0x3F800001
_NB = 128
_SEG2 = _N // (_L * 4)
_U = 4


def _process_row(row, row_v, cand_v, tie_v, hist_v, k_s, scores_hbm, out_hbm):
    pltpu.sync_copy(scores_hbm.at[row], row_v)
    iota = lax.iota(jnp.int32, _L)
    ones_i = jnp.ones((_L,), jnp.int32)
    zero_i = jnp.zeros((_L,), jnp.int32)

    @plsc.parallel_loop(0, _NB, unroll=4)
    def _(i):
        hist_v[pl.ds(i * _L, _L)] = zero_i

    @plsc.parallel_loop(
        0, _N // (_L * _U), unroll=2,
        carry=tuple(jnp.zeros((_L,), jnp.float32) for _ in range(_U)))
    def saccs(i, saccs):
        out = []
        for u in range(_U):
            x = row_v[pl.ds((i * _U + u) * _L, _L)]
            xc = jnp.clip(x, 0.0, 1.0)
            out.append(saccs[u] + xc)
            e = plsc.bitcast(1.0 - xc, jnp.int32) >> 23
            plsc.addupdate_scatter(hist_v, [(e << 4) + iota], ones_i)
        return tuple(out)

    s_row = jnp.sum(sum(saccs))

    carry = jnp.int32(0)
    bstar = jnp.int32(0)
    base = jnp.int32(0)
    for g in range(_NB // _L):
        tot = zero_i
        bidx = (g * _L + iota) << 4
        for p in range(_L):
            tot = tot + plsc.load_gather(hist_v, [bidx + p])
        cum = plsc.cumsum(tot) + carry
        carry = cum[_L - 1]
        below = cum < _K
        bstar = bstar + plsc.all_reduce_population_count(below)[0]
        base = jnp.maximum(base, jnp.max(jnp.where(below, cum, 0)))

    seg_bases = [(iota * _U + u) * _SEG2 for u in range(_U)]

    @plsc.parallel_loop(
        0, _N // (_L * _U), unroll=2,
        carry=tuple(zero_i for _ in range(_U)))
    def percnts(i, percnts):
        out = []
        for u in range(_U):
            x = row_v[pl.ds((i * _U + u) * _L, _L)]
            xc = jnp.clip(x, 0.0, 1.0)
            e = plsc.bitcast(1.0 - xc, jnp.int32) >> 23
            m = e == bstar
            plsc.store_scatter(cand_v, [seg_bases[u] + percnts[u]], xc, mask=m)
            out.append(percnts[u] + jnp.where(m, 1, 0))
        return tuple(out)

    cs_max = jnp.max(jnp.maximum(jnp.maximum(percnts[0], percnts[1]),
                                 jnp.maximum(percnts[2], percnts[3])))
    cs_tot = jnp.sum(percnts[0] + percnts[1] + percnts[2] + percnts[3])

    def bit(carryv):
        lo, hi, cnt_lo, cnt_hi = carryv
        mid = (lo + hi) >> 1

        def cbody(i, cacc):
            for u in range(_U):
                xb = plsc.bitcast(
                    plsc.load_gather(cand_v, [seg_bases[u] + i]), jnp.int32)
                valid = i < percnts[u]
                cacc = cacc + jnp.where(valid & (xb >= mid), 1, 0)
            return cacc

        cnt = jnp.sum(lax.fori_loop(0, cs_max, cbody, zero_i)) + base
        ge = cnt >= _K
        return (jnp.where(ge, mid, lo), jnp.where(ge, hi, mid),
                jnp.where(ge, cnt, cnt_lo), jnp.where(ge, cnt_hi, cnt))

    lo, hi, cnt_lo, cnt_hi = lax.while_loop(
        lambda c: c[1] - c[0] > 1, bit,
        (jnp.int32(0), jnp.int32(_HI0), base + cs_tot, jnp.int32(0)))

    t = lo
    n_gt = cnt_hi
    need = _K - n_gt
    cond = s_row > k_s

    @pl.when(cond & (cnt_lo == _K))
    def _():
        @plsc.parallel_loop(0, _N // (_L * _U), unroll=2)
        def _(i):
            for u in range(_U):
                sl = pl.ds((i * _U + u) * _L, _L)
                xc = jnp.clip(row_v[sl], 0.0, 1.0)
                xb = plsc.bitcast(xc, jnp.int32)
                cand_v[sl] = jnp.where(xb >= t, xc, 0.0)

    @pl.when(cond & (cnt_lo != _K))
    def _():
        def obody(i, wv):
            x = row_v[pl.ds(i * _L, _L)]
            xc = jnp.clip(x, 0.0, 1.0)
            xb = plsc.bitcast(xc, jnp.int32)
            meq = xb == t
            csum = plsc.cumsum(jnp.where(meq, 1, 0))
            tix = wv + csum - 1
            plsc.store_scatter(tie_v, [tix], i * _L + iota,
                               mask=meq & (tix < _K))
            cand_v[pl.ds(i * _L, _L)] = jnp.where(xb > t, xc, 0.0)
            return wv + plsc.all_reduce_population_count(meq)

        lax.fori_loop(0, _N // _L, obody, zero_i)
        tvals = plsc.bitcast(jnp.broadcast_to(t, (_L,)), jnp.float32)

        def rbody(i, c):
            tix = tie_v[pl.ds(i * _L, _L)]
            valid = (i * _L + iota) < need
            plsc.store_scatter(cand_v, [tix], tvals, mask=valid)
            return c

        lax.fori_loop(0, (need + _L - 1) // _L, rbody, 0)

    @pl.when(jnp.logical_not(cond))
    def _():
        @plsc.parallel_loop(0, _N // (_L * _U), unroll=2)
        def _(i):
            for u in range(_U):
                sl = pl.ds((i * _U + u) * _L, _L)
                cand_v[sl] = jnp.clip(row_v[sl], 0.0, 1.0)

    pltpu.sync_copy(cand_v, out_hbm.at[row])


def _make_sc_fn(n_rows):
    rows_per_worker = max(1, n_rows // 32)

    def _sc_body(scores_hbm, kvec_hbm, out_hbm, row_v, cand_v, tie_v, hist_v,
                 kv_v):
        wid = lax.axis_index("s") * 2 + lax.axis_index("c")
        pltpu.sync_copy(kvec_hbm, kv_v)
        k_s = jnp.sum(kv_v[...]) * 0.0625
        for r in range(rows_per_worker):
            _process_row(wid * rows_per_worker + r, row_v, cand_v, tie_v,
                         hist_v, k_s, scores_hbm, out_hbm)

    mesh = plsc.VectorSubcoreMesh(core_axis_name="c", subcore_axis_name="s")
    return functools.partial(
        pl.kernel,
        mesh=mesh,
        out_type=jax.ShapeDtypeStruct((n_rows, _N), jnp.float32),
        scratch_types=[
            pltpu.VMEM((_N,), jnp.float32),
            pltpu.VMEM((_N,), jnp.float32),
            pltpu.VMEM((_K + _L,), jnp.int32),
            pltpu.VMEM((_NB * _L,), jnp.int32),
            pltpu.VMEM((_L,), jnp.float32),
        ],
        compiler_params=pltpu.CompilerParams(needs_layout_passes=False),
    )(_sc_body)


def _tc_rowsum(m):
    v = m
    while v.shape[1] > 128:
        h = v.shape[1] // 2
        v = v[:, :h] + v[:, h:]
    return jnp.sum(v, axis=-1, keepdims=True)


def _tc_count(pred):
    return _tc_rowsum(pred.astype(jnp.int32))


def _tc_body(k_ref, x_ref, o_ref):
    x = x_ref[...]
    xc = jnp.clip(x, 0.0, 1.0)
    s = _tc_rowsum(xc)
    xb = lax.bitcast_convert_type(xc, jnp.int32)

    def vstep(_, carry):
        lo, hi = carry
        mid = (lo + hi) >> 1
        ge = _tc_count(xb >= mid) >= _K
        return jnp.where(ge, mid, lo), jnp.where(ge, hi, mid)

    r = x.shape[0]
    lo0 = jnp.zeros((r, 1), jnp.int32)
    hi0 = jnp.full((r, 1), _HI0, jnp.int32)
    lo, _ = lax.fori_loop(0, 31, vstep, (lo0, hi0))
    t = lo

    eq = xb == t
    n_ge = _tc_count(xb >= t)
    n_gt = _tc_count(xb > t)
    need = _K - n_gt
    idx = lax.broadcasted_iota(jnp.int32, x.shape, 1)

    def tie_bisect():
        def jstep(_, carry):
            jlo, jhi = carry
            mid = (jlo + jhi) >> 1
            geq = _tc_count(eq & (idx < mid)) >= need
            return jnp.where(geq, jlo, mid), jnp.where(geq, mid, jhi)

        jlo0 = jnp.zeros((r, 1), jnp.int32)
        jhi0 = jnp.full((r, 1), _N, jnp.int32)
        _, jhi = lax.fori_loop(0, 16, jstep, (jlo0, jhi0))
        return jhi

    jhi = lax.cond(jnp.all(n_ge == _K),
                   lambda: jnp.full((r, 1), _N, jnp.int32),
                   tie_bisect)
    mask = (xb > t) | (eq & (idx < jhi))
    cond = s > k_ref[0, 0]
    o_ref[...] = jnp.where(cond, jnp.where(mask, xc, 0.0), xc)


_TC_BLOCK = 16


def _tc_fn(x, kf):
    rows = x.shape[0]
    return pl.pallas_call(
        _tc_body,
        grid=(rows // _TC_BLOCK,),
        in_specs=[
            pl.BlockSpec(memory_space=pltpu.SMEM),
            pl.BlockSpec((_TC_BLOCK, _N), lambda i: (i, 0)),
        ],
        out_specs=pl.BlockSpec((_TC_BLOCK, _N), lambda i: (i, 0)),
        out_shape=jax.ShapeDtypeStruct(x.shape, x.dtype),
    )(kf, x)


_SC_ROWS = 32


def kernel(scores, k):
    kvec = jnp.broadcast_to(jnp.asarray(k, jnp.float32), (_L,))
    kf = jnp.asarray(k, jnp.float32).reshape(1, 1)
    rows = scores.shape[0]
    n_tc = rows - _SC_ROWS
    out_sc = _make_sc_fn(_SC_ROWS)(scores[n_tc:], kvec)
    out_tc = _tc_fn(scores[:n_tc], kf)
    return jnp.concatenate([out_tc, out_sc], axis=0)
